# baseline (speedup 1.0000x reference)
import jax
import jax.numpy as jnp
from jax import lax
from jax.experimental import pallas as pl
from jax.experimental.pallas import tpu as pltpu
from jax.experimental.pallas import tpu_sc as plsc

N = 320000
D = 128
S = 10000
NC = 2
NS = 16
NW = NC * NS
SEG_W = 320
T = SEG_W + 1
DUMP = SEG_W
C = 48
IDSB = 64
NB = N // 16

_mesh = plsc.VectorSubcoreMesh(core_axis_name="c", subcore_axis_name="s")


def _body(x_hbm, ids_hbm, out_hbm, acc, cnt, xb0, xb1, xb2, ib0, ib1, ib2,
          pbuf, sem0, sem1, sem2, st0, st1, st2):
    cid = lax.axis_index("c")
    sid = lax.axis_index("s")
    w = cid * NS + sid

    zero16 = jnp.zeros((16,), jnp.float32)
    one16 = jnp.full((16,), 1.0, jnp.float32)

    def zrow(t, carry):
        for j in range(D // 16):
            acc[t, pl.ds(j * 16, 16)] = zero16
        cnt[t, :] = zero16
        return carry
    lax.fori_loop(0, T, zrow, 0)

    def lower_bound(target):
        def bs(it, lohi):
            lo, hi = lohi
            mid = (lo + hi) // 2
            pltpu.sync_copy(ids_hbm.at[pl.ds(mid * 16, 16)], pbuf)
            bmax = pbuf[pl.ds(0, 16)][15]
            below = bmax < target
            return (jnp.where(below, mid + 1, lo), jnp.where(below, hi, mid))
        b, _ = lax.fori_loop(0, 15, bs, (0, NB))
        bc = jnp.minimum(b, NB - 1)
        pltpu.sync_copy(ids_hbm.at[pl.ds(bc * 16, 16)], pbuf)
        v = pbuf[pl.ds(0, 16)]
        f = jnp.int32(16)
        for l in reversed(range(16)):
            f = jnp.where(v[l] >= target, jnp.int32(l), f)
        return jnp.where(b >= NB, N, bc * 16 + f)

    lo_row = lower_bound(w * SEG_W)
    hi_row = lower_bound((w + 1) * SEG_W)
    nrows = hi_row - lo_row
    nfull = nrows // C
    ntail = nrows - nfull * C

    def cstart_of(g):
        return jnp.minimum(lo_row + g * C, N - C)

    def aoff_of(cs):
        return jnp.minimum(pl.multiple_of((cs // 8) * 8, 8), N - IDSB)

    def vlo_of(g):
        return jnp.minimum(lo_row + g * C, hi_row)

    def issue_load(g, xb, ib, sem):
        cs = cstart_of(g)
        pltpu.async_copy(x_hbm.at[pl.ds(cs * D, C * D)], xb, sem)
        pltpu.async_copy(ids_hbm.at[pl.ds(aoff_of(cs), IDSB)], ib, sem)

    def wait_load(xb, ib, sem):
        pltpu.make_async_copy(x_hbm.at[pl.ds(0, C * D)], xb, sem).wait()
        pltpu.make_async_copy(ids_hbm.at[pl.ds(0, IDSB)], ib, sem).wait()

    BUFS = ((xb0, ib0, sem0), (xb1, ib1, sem1), (xb2, ib2, sem2))
    STS = (st0, st1, st2)

    def wait_store(b):
        pltpu.make_async_copy(BUFS[b][0], out_hbm.at[pl.ds(0, C * D)],
                              STS[b]).wait()

    NJ = D // 16

    def flush(pt, svec, cvec):
        for j in range(NJ):
            sl = pl.ds(j * 16, 16)
            acc[pt, sl] = acc[pt, sl] + svec[j]
        cnt[pt, :] = cnt[pt, :] + cvec

    def accum_chunk(xbuf, idsbuf, cstart, skew, vlo, vhi, carry):
        def grp(k, car):
            prev_t = car[0]
            c16 = car[1]
            s = list(car[2:])
            vec = idsbuf[pl.ds(skew + k * 16, 16)]
            for l in range(16):
                i = k * 16 + l
                seg = vec[l]
                ridx = cstart + i
                valid = (ridx >= vlo) & (ridx < vhi)
                t = jnp.where(valid, seg - w * SEG_W, DUMP)
                change = t != prev_t
                sl_ = list(s)
                pt_ = prev_t
                cc_ = c16

                @pl.when(change)
                def _():
                    flush(pt_, sl_, cc_)
                keep = jnp.where(change, jnp.float32(0), jnp.float32(1))
                keep16 = jnp.broadcast_to(keep, (16,))
                xrow = [xbuf[pl.ds(i * D + j * 16, 16)] for j in range(NJ)]
                s = [s[j] * keep16 + xrow[j] for j in range(NJ)]
                c16 = c16 * keep16 + one16
                prev_t = t
            return (prev_t, c16, *s)
        return lax.fori_loop(0, C // 16, grp, carry)

    def out_chunk(xbuf, idsbuf, cstart, skew, carry):
        def grp(k, car):
            vec = idsbuf[pl.ds(skew + k * 16, 16)]
            same = vec[0] == vec[15]

            @pl.when(same)
            def _():
                t = jnp.clip(vec[0] - w * SEG_W, 0, DUMP)
                m = [acc[t, pl.ds(j * 16, 16)] for j in range(NJ)]
                for l in range(16):
                    i = k * 16 + l
                    for j in range(NJ):
                        sl = pl.ds(i * D + j * 16, 16)
                        xbuf[sl] = xbuf[sl] - m[j]

            @pl.when(jnp.logical_not(same))
            def _():
                for l in range(16):
                    i = k * 16 + l
                    t = jnp.clip(vec[l] - w * SEG_W, 0, DUMP)
                    for j in range(NJ):
                        sl = pl.ds(i * D + j * 16, 16)
                        xbuf[sl] = xbuf[sl] - acc[t, pl.ds(j * 16, 16)]
            return car
        return lax.fori_loop(0, C // 16, grp, carry)

    nchunks = nfull + 1
    npairs = (nchunks + 1) // 2

    carry = (jnp.int32(DUMP), zero16) + tuple([zero16] * NJ)
    issue_load(0, *BUFS[0])
    issue_load(1, *BUFS[1])

    def acc_pair(p, car):
        for b in range(2):
            xb, ib, sem = BUFS[b]
            g = 2 * p + b
            wait_load(xb, ib, sem)
            cs = cstart_of(g)
            skew = cs - aoff_of(cs)
            car = accum_chunk(xb, ib, cs, skew, vlo_of(g), vlo_of(g + 1),
                              car)
            issue_load(g + 2, xb, ib, sem)
        return car
    carry = lax.fori_loop(0, npairs, acc_pair, carry)
    flush(carry[0], list(carry[2:]), carry[1])
    wait_load(*BUFS[0])
    wait_load(*BUFS[1])

    def mean_row(t, carry):
        inv = one16 / jnp.maximum(cnt[t, :], one16)
        for j in range(D // 16):
            sl = pl.ds(j * 16, 16)
            acc[t, sl] = acc[t, sl] * inv
        return carry
    lax.fori_loop(0, SEG_W, mean_row, 0)

    ntriples = (nchunks + 2) // 3

    def was_full(g):
        return (vlo_of(g + 1) - vlo_of(g)) == C

    issue_load(0, *BUFS[0])
    issue_load(1, *BUFS[1])

    def out_triple(p, car):
        for b in range(3):
            xb, ib, sem = BUFS[b]
            g = 3 * p + b
            wait_load(xb, ib, sem)
            cs = cstart_of(g)
            skew = cs - aoff_of(cs)
            out_chunk(xb, ib, cs, skew, 0)
            vlo = vlo_of(g)
            wlen = vlo_of(g + 1) - vlo

            @pl.when(wlen == C)
            def _():
                pltpu.async_copy(xb, out_hbm.at[pl.ds(vlo * D, C * D)],
                                 STS[b])

            @pl.when(wlen < C)
            def _():
                shift = vlo - cs

                def wrow(r, car2):
                    pltpu.sync_copy(
                        xb.at[pl.ds((shift + r) * D, D)],
                        out_hbm.at[pl.ds((vlo + r) * D, D)])
                    return car2
                lax.fori_loop(0, wlen, wrow, 0)

            b2 = (b + 2) % 3

            @pl.when((g >= 1) & was_full(g - 1))
            def _():
                wait_store(b2)
            issue_load(g + 2, *BUFS[b2])
        return car
    lax.fori_loop(0, ntriples, out_triple, 0)
    wait_load(*BUFS[0])
    wait_load(*BUFS[1])
    @pl.when(was_full(3 * ntriples - 1))
    def _():
        wait_store(2)


_sc_kernel = pl.kernel(
    _body,
    out_type=jax.ShapeDtypeStruct((N * D,), jnp.float32),
    mesh=_mesh,
    scratch_types=[
        pltpu.VMEM((T, D), jnp.float32),
        pltpu.VMEM((T, 16), jnp.float32),
        pltpu.VMEM((C * D,), jnp.float32),
        pltpu.VMEM((C * D,), jnp.float32),
        pltpu.VMEM((C * D,), jnp.float32),
        pltpu.VMEM((IDSB,), jnp.int32),
        pltpu.VMEM((IDSB,), jnp.int32),
        pltpu.VMEM((IDSB,), jnp.int32),
        pltpu.VMEM((16,), jnp.int32),
        pltpu.SemaphoreType.DMA,
        pltpu.SemaphoreType.DMA,
        pltpu.SemaphoreType.DMA,
        pltpu.SemaphoreType.DMA,
        pltpu.SemaphoreType.DMA,
        pltpu.SemaphoreType.DMA,
    ],
)


def kernel(x_pos, segment_ids):
    ids = segment_ids.astype(jnp.int32)
    out_flat = _sc_kernel(x_pos.reshape(N * D), ids)
    return out_flat.reshape(N, D)


# speedup vs baseline: 1.2325x; 1.2325x over previous
import jax
import jax.numpy as jnp
from jax import lax
from jax.experimental import pallas as pl
from jax.experimental.pallas import tpu as pltpu
from jax.experimental.pallas import tpu_sc as plsc

N = 320000
D = 128
S = 10000
NC = 2
NS = 16
NW = NC * NS
SEG_W = 320
T = SEG_W + 1
DUMP = SEG_W
C = 64
IDSB = 80
NB = N // 16

_mesh = plsc.VectorSubcoreMesh(core_axis_name="c", subcore_axis_name="s")


def _body(x_hbm, ids_hbm, out_hbm, acc, cnt, xb0, xb1, ib0, ib1, pbuf,
          sem0, sem1):
    cid = lax.axis_index("c")
    sid = lax.axis_index("s")
    w = cid * NS + sid

    zero16 = jnp.zeros((16,), jnp.float32)
    one16 = jnp.full((16,), 1.0, jnp.float32)

    def zrow(t, carry):
        for j in range(D // 16):
            acc[t, pl.ds(j * 16, 16)] = zero16
        cnt[t, :] = zero16
        return carry
    lax.fori_loop(0, T, zrow, 0)

    def lower_bound(target):
        def bs(it, lohi):
            lo, hi = lohi
            mid = (lo + hi) // 2
            pltpu.sync_copy(ids_hbm.at[pl.ds(mid * 16, 16)], pbuf)
            bmax = pbuf[pl.ds(0, 16)][15]
            below = bmax < target
            return (jnp.where(below, mid + 1, lo), jnp.where(below, hi, mid))
        b, _ = lax.fori_loop(0, 15, bs, (0, NB))
        bc = jnp.minimum(b, NB - 1)
        pltpu.sync_copy(ids_hbm.at[pl.ds(bc * 16, 16)], pbuf)
        v = pbuf[pl.ds(0, 16)]
        f = jnp.int32(16)
        for l in reversed(range(16)):
            f = jnp.where(v[l] >= target, jnp.int32(l), f)
        return jnp.where(b >= NB, N, bc * 16 + f)

    lo_row = lower_bound(w * SEG_W)
    hi_row = lower_bound((w + 1) * SEG_W)
    nrows = hi_row - lo_row
    nfull = nrows // C
    ntail = nrows - nfull * C

    def cstart_of(g):
        return jnp.minimum(lo_row + g * C, N - C)

    def aoff_of(cs):
        return jnp.minimum(pl.multiple_of((cs // 8) * 8, 8), N - IDSB)

    def vlo_of(g):
        return jnp.minimum(lo_row + g * C, hi_row)

    def issue_load(g, xb, ib, sem):
        cs = cstart_of(g)
        pltpu.async_copy(x_hbm.at[pl.ds(cs * D, C * D)], xb, sem)
        pltpu.async_copy(ids_hbm.at[pl.ds(aoff_of(cs), IDSB)], ib, sem)

    def wait_load(xb, ib, sem):
        pltpu.make_async_copy(x_hbm.at[pl.ds(0, C * D)], xb, sem).wait()
        pltpu.make_async_copy(ids_hbm.at[pl.ds(0, IDSB)], ib, sem).wait()

    BUFS = ((xb0, ib0, sem0), (xb1, ib1, sem1))

    NJ = D // 16

    def flush(pt, svec, cvec):
        for j in range(NJ):
            sl = pl.ds(j * 16, 16)
            acc[pt, sl] = acc[pt, sl] + svec[j]
        cnt[pt, :] = cnt[pt, :] + cvec

    def accum_chunk(xbuf, idsbuf, cstart, skew, vlo, vhi, carry):
        def grp(k, car):
            prev_t = car[0]
            c16 = car[1]
            s = list(car[2:])
            vec = idsbuf[pl.ds(skew + k * 16, 16)]
            for l in range(16):
                i = k * 16 + l
                seg = vec[l]
                ridx = cstart + i
                valid = (ridx >= vlo) & (ridx < vhi)
                t = jnp.where(valid, seg - w * SEG_W, DUMP)
                change = t != prev_t
                sl_ = list(s)
                pt_ = prev_t
                cc_ = c16

                @pl.when(change)
                def _():
                    flush(pt_, sl_, cc_)
                keep = jnp.where(change, jnp.float32(0), jnp.float32(1))
                keep16 = jnp.broadcast_to(keep, (16,))
                xrow = [xbuf[pl.ds(i * D + j * 16, 16)] for j in range(NJ)]
                s = [s[j] * keep16 + xrow[j] for j in range(NJ)]
                c16 = c16 * keep16 + one16
                prev_t = t
            return (prev_t, c16, *s)
        return lax.fori_loop(0, C // 16, grp, carry)

    def out_chunk(xbuf, idsbuf, cstart, skew, carry):
        def grp(k, car):
            vec = idsbuf[pl.ds(skew + k * 16, 16)]
            same = vec[0] == vec[15]

            @pl.when(same)
            def _():
                t = jnp.clip(vec[0] - w * SEG_W, 0, DUMP)
                m = [acc[t, pl.ds(j * 16, 16)] for j in range(NJ)]
                for l in range(16):
                    i = k * 16 + l
                    for j in range(NJ):
                        sl = pl.ds(i * D + j * 16, 16)
                        xbuf[sl] = xbuf[sl] - m[j]

            @pl.when(jnp.logical_not(same))
            def _():
                for l in range(16):
                    i = k * 16 + l
                    t = jnp.clip(vec[l] - w * SEG_W, 0, DUMP)
                    for j in range(NJ):
                        sl = pl.ds(i * D + j * 16, 16)
                        xbuf[sl] = xbuf[sl] - acc[t, pl.ds(j * 16, 16)]
            return car
        return lax.fori_loop(0, C // 16, grp, carry)

    nchunks = nfull + 1
    npairs = (nchunks + 1) // 2

    carry = (jnp.int32(DUMP), zero16) + tuple([zero16] * NJ)
    issue_load(0, *BUFS[0])
    issue_load(1, *BUFS[1])

    def acc_pair(p, car):
        for b in range(2):
            xb, ib, sem = BUFS[b]
            g = 2 * p + b
            wait_load(xb, ib, sem)
            cs = cstart_of(g)
            skew = cs - aoff_of(cs)
            car = accum_chunk(xb, ib, cs, skew, vlo_of(g), vlo_of(g + 1),
                              car)
            issue_load(g + 2, xb, ib, sem)
        return car
    carry = lax.fori_loop(0, npairs, acc_pair, carry)
    flush(carry[0], list(carry[2:]), carry[1])
    wait_load(*BUFS[0])
    wait_load(*BUFS[1])

    def mean_row(t, carry):
        inv = one16 / jnp.maximum(cnt[t, :], one16)
        for j in range(D // 16):
            sl = pl.ds(j * 16, 16)
            acc[t, sl] = acc[t, sl] * inv
        return carry
    lax.fori_loop(0, SEG_W, mean_row, 0)

    issue_load(0, *BUFS[0])
    issue_load(1, *BUFS[1])

    def out_pair(p, car):
        for b in range(2):
            xb, ib, sem = BUFS[b]
            g = 2 * p + b
            wait_load(xb, ib, sem)
            cs = cstart_of(g)
            skew = cs - aoff_of(cs)
            out_chunk(xb, ib, cs, skew, 0)
            vlo = vlo_of(g)
            wlen = vlo_of(g + 1) - vlo

            @pl.when(wlen == C)
            def _():
                pltpu.sync_copy(xb, out_hbm.at[pl.ds(vlo * D, C * D)])

            @pl.when(wlen < C)
            def _():
                shift = vlo - cs

                def wrow(r, car2):
                    pltpu.sync_copy(
                        xb.at[pl.ds((shift + r) * D, D)],
                        out_hbm.at[pl.ds((vlo + r) * D, D)])
                    return car2
                lax.fori_loop(0, wlen, wrow, 0)
            issue_load(g + 2, xb, ib, sem)
        return car
    lax.fori_loop(0, npairs, out_pair, 0)
    wait_load(*BUFS[0])
    wait_load(*BUFS[1])


_sc_kernel = pl.kernel(
    _body,
    out_type=jax.ShapeDtypeStruct((N * D,), jnp.float32),
    mesh=_mesh,
    scratch_types=[
        pltpu.VMEM((T, D), jnp.float32),
        pltpu.VMEM((T, 16), jnp.float32),
        pltpu.VMEM((C * D,), jnp.float32),
        pltpu.VMEM((C * D,), jnp.float32),
        pltpu.VMEM((IDSB,), jnp.int32),
        pltpu.VMEM((IDSB,), jnp.int32),
        pltpu.VMEM((16,), jnp.int32),
        pltpu.SemaphoreType.DMA,
        pltpu.SemaphoreType.DMA,
    ],
)


def kernel(x_pos, segment_ids):
    ids = segment_ids.astype(jnp.int32)
    out_flat = _sc_kernel(x_pos.reshape(N * D), ids)
    return out_flat.reshape(N, D)
